# baseline (speedup 1.0000x reference)
import functools

import jax
import jax.numpy as jnp
import numpy as np
from jax import lax
from jax.experimental import pallas as pl
from jax.experimental.pallas import tpu as pltpu
from jax.experimental.pallas import tpu_sc as plsc

N = 10000
E = 320000
D = 128
H = 128
C = 40
CP = 40

NC = 2
NS = 16
NW = NC * NS
CH = 128
NCHUNK = 80
EPT = NCHUNK * CH
E_PAD = NW * EPT
N_PAD = 10240
ROWS_T = N_PAD // NS
TRASH = N

_MESH = plsc.VectorSubcoreMesh(core_axis_name="c", subcore_axis_name="s")
_SC_PARAMS = pltpu.CompilerParams(use_tc_tiling_on_sc=False)


_FULL = N // ROWS_T
_LAST = N - _FULL * ROWS_T


def _drain_stripes(c, s, acc, outA, outB):
    def drain(out, rows, base):
        pltpu.sync_copy(acc.at[pl.ds(base, rows)], out.at[pl.ds(base, rows)])

    for cc, out in ((0, outA), (1, outB)):
        @pl.when(jnp.logical_and(c == cc, s < _FULL))
        def _(out=out):
            drain(out, ROWS_T, s * ROWS_T)
        if _LAST:
            @pl.when(jnp.logical_and(c == cc, s == _FULL))
            def _(out=out):
                drain(out, _LAST, _FULL * ROWS_T)


NHALF = 2
CPH = NCHUNK // NHALF


def _sc_agg(width, ch=CH, nb=2, nstage=NHALF, params=_SC_PARAMS):
    nchunk = EPT // ch
    cps = nchunk // nstage
    ng = cps // nb

    @functools.partial(
        pl.kernel, mesh=_MESH, compiler_params=params,
        out_type=[jax.ShapeDtypeStruct((N, width), jnp.float32),
                  jax.ShapeDtypeStruct((N, width), jnp.float32)],
        scratch_types=[pltpu.VMEM((cps, ch), jnp.int32) for _ in range(4)]
        + [pltpu.VMEM((ch, width), jnp.float32) for _ in range(nb)]
        + [pltpu.VMEM_SHARED((N_PAD, width), jnp.float32)]
        + [pltpu.SemaphoreType.DMA] * (2 * nb + 2))
    def body(feat_hbm, src_hbm, dst_hbm, zrows_hbm, outA, outB, *rest):
        src_b = rest[0:2]
        dst_b = rest[2:4]
        rows = rest[4:4 + nb]
        acc_sh = rest[4 + nb]
        gsem = rest[5 + nb:5 + 2 * nb]
        ssem = rest[5 + 2 * nb:5 + 3 * nb]
        isem = rest[5 + 3 * nb:]
        c = lax.axis_index("c")
        s = lax.axis_index("s")
        wid = c * NS + s

        pltpu.sync_copy(zrows_hbm, acc_sh.at[pl.ds(s * ROWS_T, ROWS_T)])
        plsc.subcore_barrier()

        def prefetch(h, p):
            base = wid * nchunk + h * cps
            pltpu.async_copy(src_hbm.at[pl.ds(base, cps)], src_b[p], isem[p])
            pltpu.async_copy(dst_hbm.at[pl.ds(base, cps)], dst_b[p], isem[p])

        def wait_prefetch(p):
            pltpu.make_async_copy(src_hbm.at[pl.ds(0, cps)], src_b[p],
                                  isem[p]).wait()
            pltpu.make_async_copy(dst_hbm.at[pl.ds(0, cps)], dst_b[p],
                                  isem[p]).wait()

        def gather(j, b, p):
            pltpu.async_copy(feat_hbm.at[src_b[p].at[j]], rows[b], gsem[b])

        def scatter(j, b, p):
            pltpu.async_copy(rows[b], acc_sh.at[dst_b[p].at[j]], ssem[b],
                             add=True)

        def wait_gather(b):
            pltpu.make_async_copy(feat_hbm.at[src_b[0].at[0]], rows[b],
                                  gsem[b]).wait()

        def wait_scatter(b):
            pltpu.make_async_copy(rows[b], acc_sh.at[dst_b[0].at[0]],
                                  ssem[b]).wait()

        def run_stage(h, p):
            wait_prefetch(p)
            for b in range(nb):
                gather(b, b, p)

            def group(g, carry2):
                for b in range(nb):
                    wait_gather(b)
                    scatter(g * nb + b, b, p)
                for b in range(nb):
                    jn = (g + 1) * nb + b

                    @pl.when(jn < cps)
                    def _(jn=jn, b=b):
                        wait_scatter(b)
                        gather(jn, b, p)
                return carry2

            lax.fori_loop(0, ng, group, 0)
            for b in range(nb):
                wait_scatter(b)

        prefetch(0, 0)
        prefetch(1, 1)

        def pair(g, carry):
            for p in range(2):
                h = 2 * g + p
                run_stage(h, p)

                @pl.when(h + 2 < nstage)
                def _(h=h, p=p):
                    prefetch(h + 2, p)
            return carry

        lax.fori_loop(0, nstage // 2, pair, 0)

        plsc.subcore_barrier()
        _drain_stripes(c, s, acc_sh, outA, outB)

    return body


@functools.partial(
    pl.kernel, mesh=_MESH, compiler_params=_SC_PARAMS,
    out_type=[jax.ShapeDtypeStruct((N, 16), jnp.float32),
              jax.ShapeDtypeStruct((N, 16), jnp.float32)],
    scratch_types=[
        pltpu.VMEM((NCHUNK, CH), jnp.int32),
        pltpu.VMEM((CH, 16), jnp.float32),
        pltpu.VMEM_SHARED((N_PAD, 16), jnp.float32),
        pltpu.SemaphoreType.DMA,
    ])
def _sc_deg(dst_hbm, ones_hbm, zdeg_hbm, degA, degB, dst_v, ones_v, deg_sh,
            sem):
    c = lax.axis_index("c")
    s = lax.axis_index("s")
    wid = c * NS + s
    GRP = 8

    pltpu.sync_copy(zdeg_hbm, deg_sh.at[pl.ds(s * ROWS_T, ROWS_T)])
    pltpu.sync_copy(ones_hbm, ones_v)
    pltpu.sync_copy(dst_hbm.at[pl.ds(wid * NCHUNK, NCHUNK)], dst_v)

    plsc.subcore_barrier()

    def fire(g):
        for b in range(GRP):
            pltpu.async_copy(ones_v, deg_sh.at[dst_v.at[g * GRP + b]], sem,
                             add=True)

    def drain():
        for _ in range(GRP):
            pltpu.make_async_copy(ones_v, deg_sh.at[dst_v.at[0]],
                                  sem).wait()

    fire(0)

    def group(g, carry):
        fire(g)
        drain()
        return carry

    lax.fori_loop(1, NCHUNK // GRP, group, 0)
    drain()

    plsc.subcore_barrier()
    _drain_stripes(c, s, deg_sh, degA, degB)


CH48 = 64
_sc_agg128 = _sc_agg(D, ch=CH48, nb=4, nstage=8)
_sc_agg48 = _sc_agg(CP, ch=CH48, nb=4, nstage=8)

BLK = 2000


def _selfproj_body(x, W, b, out):
    dn = (((1,), (1,)), ((), ()))
    out[...] = lax.dot_general(x[...], W[...], dn,
                               preferred_element_type=jnp.float32) + b[...]


def _dense0_body(aggA, aggB, degA, degB, xr, W0l, W1lp, h_out, hW_out):
    deg = jnp.maximum(degA[:, :1] + degB[:, :1], 1.0)
    mean = (aggA[...] + aggB[...]) / deg
    dn = (((1,), (1,)), ((), ()))
    h = lax.dot_general(mean, W0l[...], dn, preferred_element_type=jnp.float32)
    h = jnp.maximum(h + xr[...], 0.0)
    h_out[...] = h
    hW_out[...] = lax.dot_general(h, W1lp[...], dn,
                                  preferred_element_type=jnp.float32)


def _final_body(zref, out):
    z = zref[...]
    m = jnp.max(z, axis=1, keepdims=True)
    lse = jnp.log(jnp.sum(jnp.exp(z - m), axis=1, keepdims=True)) + m
    out[...] = z - lse


def _row_spec(width):
    return pl.BlockSpec((BLK, width), lambda i: (i, 0))


def _full_spec(shape):
    return pl.BlockSpec(shape, lambda i: (0,) * len(shape))


def _selfproj(width):
    return pl.pallas_call(
        _selfproj_body,
        grid=(N // BLK,),
        in_specs=[_row_spec(H), _full_spec((width, H)),
                  _full_spec((1, width))],
        out_specs=_row_spec(width),
        out_shape=jax.ShapeDtypeStruct((N, width), jnp.float32),
    )


_selfproj128 = _selfproj(H)
_selfproj48 = _selfproj(CP)

_dense0 = pl.pallas_call(
    _dense0_body,
    grid=(N // BLK,),
    in_specs=[
        _row_spec(D), _row_spec(D),
        _row_spec(16), _row_spec(16),
        _row_spec(H),
        _full_spec((H, D)), _full_spec((CP, H)),
    ],
    out_specs=[_row_spec(H), _row_spec(CP)],
    out_shape=[jax.ShapeDtypeStruct((N, H), jnp.float32),
               jax.ShapeDtypeStruct((N, CP), jnp.float32)],
)

_final = pl.pallas_call(
    _final_body,
    grid=(N // BLK,),
    in_specs=[_row_spec(CP)],
    out_specs=_row_spec(CP),
    out_shape=jax.ShapeDtypeStruct((N, CP), jnp.float32),
)


_PAD = E_PAD - E
_SRC_PAD = np.arange(_PAD, dtype=np.int32) * 37 % N
_DST_PAD = (TRASH + np.arange(_PAD, dtype=np.int32) % (N_PAD - N)).astype(
    np.int32)
_ZROWS128 = np.zeros((ROWS_T, D), np.float32)
_ZROWS48 = np.zeros((ROWS_T, CP), np.float32)
_ZDEG = np.zeros((ROWS_T, 16), np.float32)
_ONES_ROWS = np.ones((CH, 16), np.float32)


def kernel(x, edge_index, y, W0_l, b0, W0_r, W1_l, b1, W1_r):
    src1 = jnp.concatenate([edge_index[0], _SRC_PAD]).reshape(
        E_PAD // CH, CH)
    dst1 = jnp.concatenate([edge_index[1], _DST_PAD]).reshape(
        E_PAD // CH, CH)

    src64 = src1.reshape(E_PAD // CH48, CH48)
    dst64 = dst1.reshape(E_PAD // CH48, CH48)

    xr = _selfproj128(x, W0_r, b0.reshape(1, H))
    degA, degB = _sc_deg(dst1, _ONES_ROWS, _ZDEG)
    aggA, aggB = _sc_agg128(x, src64, dst64, _ZROWS128)
    h, hW = _dense0(aggA, aggB, degA, degB, xr, W0_l, W1_l)
    hr = _selfproj48(h, W1_r, b1.reshape(1, C))
    agg1A, agg1B = _sc_agg48(hW, src64, dst64, _ZROWS48)
    rdeg = 1.0 / jnp.maximum(degA[:, :1] + degB[:, :1], 1.0)
    z = jnp.maximum((agg1A + agg1B) * rdeg + hr, 0.0)
    return _final(z)


# speedup vs baseline: 1.0227x; 1.0227x over previous
import functools

import jax
import jax.numpy as jnp
import numpy as np
from jax import lax
from jax.experimental import pallas as pl
from jax.experimental.pallas import tpu as pltpu
from jax.experimental.pallas import tpu_sc as plsc

N = 10000
E = 320000
D = 128
H = 128
C = 40
CP = 40

NC = 2
NS = 16
NW = NC * NS
CH = 128
NCHUNK = 80
EPT = NCHUNK * CH
E_PAD = NW * EPT
N_PAD = 10240
ROWS_T = N_PAD // NS
TRASH = N

_MESH = plsc.VectorSubcoreMesh(core_axis_name="c", subcore_axis_name="s")
_SC_PARAMS = pltpu.CompilerParams(use_tc_tiling_on_sc=False)


_FULL = N // ROWS_T
_LAST = N - _FULL * ROWS_T


def _drain_stripes(c, s, acc, outA, outB):
    def drain(out, rows, base):
        pltpu.sync_copy(acc.at[pl.ds(base, rows)], out.at[pl.ds(base, rows)])

    for cc, out in ((0, outA), (1, outB)):
        @pl.when(jnp.logical_and(c == cc, s < _FULL))
        def _(out=out):
            drain(out, ROWS_T, s * ROWS_T)
        if _LAST:
            @pl.when(jnp.logical_and(c == cc, s == _FULL))
            def _(out=out):
                drain(out, _LAST, _FULL * ROWS_T)


NHALF = 2
CPH = NCHUNK // NHALF


def _sc_agg(width, ch=CH, nb=2, nstage=NHALF, params=_SC_PARAMS):
    nchunk = EPT // ch
    cps = nchunk // nstage
    ng = cps // nb

    @functools.partial(
        pl.kernel, mesh=_MESH, compiler_params=params,
        out_type=[jax.ShapeDtypeStruct((N, width), jnp.float32),
                  jax.ShapeDtypeStruct((N, width), jnp.float32)],
        scratch_types=[pltpu.VMEM((cps, ch), jnp.int32) for _ in range(4)]
        + [pltpu.VMEM((ch, width), jnp.float32) for _ in range(nb)]
        + [pltpu.VMEM_SHARED((N_PAD, width), jnp.float32)]
        + [pltpu.SemaphoreType.DMA] * (2 * nb + 2))
    def body(feat_hbm, src_hbm, dst_hbm, zrows_hbm, outA, outB, *rest):
        src_b = rest[0:2]
        dst_b = rest[2:4]
        rows = rest[4:4 + nb]
        acc_sh = rest[4 + nb]
        gsem = rest[5 + nb:5 + 2 * nb]
        ssem = rest[5 + 2 * nb:5 + 3 * nb]
        isem = rest[5 + 3 * nb:]
        c = lax.axis_index("c")
        s = lax.axis_index("s")
        wid = c * NS + s

        pltpu.sync_copy(zrows_hbm, acc_sh.at[pl.ds(s * ROWS_T, ROWS_T)])
        plsc.subcore_barrier()

        def prefetch(h, p):
            base = wid * nchunk + h * cps
            pltpu.async_copy(src_hbm.at[pl.ds(base, cps)], src_b[p], isem[p])
            pltpu.async_copy(dst_hbm.at[pl.ds(base, cps)], dst_b[p], isem[p])

        def wait_prefetch(p):
            pltpu.make_async_copy(src_hbm.at[pl.ds(0, cps)], src_b[p],
                                  isem[p]).wait()
            pltpu.make_async_copy(dst_hbm.at[pl.ds(0, cps)], dst_b[p],
                                  isem[p]).wait()

        def gather(j, b, p):
            pltpu.async_copy(feat_hbm.at[src_b[p].at[j]], rows[b], gsem[b])

        def scatter(j, b, p):
            pltpu.async_copy(rows[b], acc_sh.at[dst_b[p].at[j]], ssem[b],
                             add=True)

        def wait_gather(b):
            pltpu.make_async_copy(feat_hbm.at[src_b[0].at[0]], rows[b],
                                  gsem[b]).wait()

        def wait_scatter(b):
            pltpu.make_async_copy(rows[b], acc_sh.at[dst_b[0].at[0]],
                                  ssem[b]).wait()

        def run_stage(h, p):
            wait_prefetch(p)
            for b in range(nb):
                gather(b, b, p)

            def group(g, carry2):
                for b in range(nb):
                    wait_gather(b)
                    scatter(g * nb + b, b, p)
                for b in range(nb):
                    jn = (g + 1) * nb + b

                    @pl.when(jn < cps)
                    def _(jn=jn, b=b):
                        wait_scatter(b)
                        gather(jn, b, p)
                return carry2

            lax.fori_loop(0, ng, group, 0)
            for b in range(nb):
                wait_scatter(b)

        prefetch(0, 0)
        prefetch(1, 1)

        def pair(g, carry):
            for p in range(2):
                h = 2 * g + p
                run_stage(h, p)

                @pl.when(h + 2 < nstage)
                def _(h=h, p=p):
                    prefetch(h + 2, p)
            return carry

        lax.fori_loop(0, nstage // 2, pair, 0)

        plsc.subcore_barrier()
        _drain_stripes(c, s, acc_sh, outA, outB)

    return body


@functools.partial(
    pl.kernel, mesh=_MESH, compiler_params=_SC_PARAMS,
    out_type=[jax.ShapeDtypeStruct((N, 16), jnp.float32),
              jax.ShapeDtypeStruct((N, 16), jnp.float32)],
    scratch_types=[
        pltpu.VMEM((NCHUNK, CH), jnp.int32),
        pltpu.VMEM((CH, 16), jnp.float32),
        pltpu.VMEM_SHARED((N_PAD, 16), jnp.float32),
        pltpu.SemaphoreType.DMA,
    ])
def _sc_deg(dst_hbm, ones_hbm, zdeg_hbm, degA, degB, dst_v, ones_v, deg_sh,
            sem):
    c = lax.axis_index("c")
    s = lax.axis_index("s")
    wid = c * NS + s
    GRP = 8

    pltpu.sync_copy(zdeg_hbm, deg_sh.at[pl.ds(s * ROWS_T, ROWS_T)])
    pltpu.sync_copy(ones_hbm, ones_v)
    pltpu.sync_copy(dst_hbm.at[pl.ds(wid * NCHUNK, NCHUNK)], dst_v)

    plsc.subcore_barrier()

    def fire(g):
        for b in range(GRP):
            pltpu.async_copy(ones_v, deg_sh.at[dst_v.at[g * GRP + b]], sem,
                             add=True)

    def drain():
        for _ in range(GRP):
            pltpu.make_async_copy(ones_v, deg_sh.at[dst_v.at[0]],
                                  sem).wait()

    fire(0)

    def group(g, carry):
        fire(g)
        drain()
        return carry

    lax.fori_loop(1, NCHUNK // GRP, group, 0)
    drain()

    plsc.subcore_barrier()
    _drain_stripes(c, s, deg_sh, degA, degB)


CH48 = 64
_sc_agg128 = _sc_agg(D, ch=CH48, nb=4, nstage=8)
_sc_agg48 = _sc_agg(CP, ch=CH48, nb=4, nstage=8)

BLK = 2000


def _selfproj_body(x, W, b, out):
    dn = (((1,), (1,)), ((), ()))
    out[...] = lax.dot_general(x[...], W[...], dn,
                               preferred_element_type=jnp.float32) + b[...]


def _dense0_body(aggA, aggB, degA, degB, xr, W0l, W1lp, h_out, hW_out):
    deg = jnp.maximum(degA[:, :1] + degB[:, :1], 1.0)
    mean = (aggA[...] + aggB[...]) / deg
    dn = (((1,), (1,)), ((), ()))
    h = lax.dot_general(mean, W0l[...], dn, preferred_element_type=jnp.float32)
    h = jnp.maximum(h + xr[...], 0.0)
    h_out[...] = h
    hW_out[...] = lax.dot_general(h, W1lp[...], dn,
                                  preferred_element_type=jnp.float32)


def _final_body(aggA, aggB, degA, degB, hr, out):
    deg = jnp.maximum(degA[:, :1] + degB[:, :1], 1.0)
    mean = (aggA[...] + aggB[...]) / deg
    z = jnp.maximum(mean + hr[...], 0.0)
    m = jnp.max(z, axis=1, keepdims=True)
    lse = jnp.log(jnp.sum(jnp.exp(z - m), axis=1, keepdims=True)) + m
    out[...] = z - lse


def _row_spec(width):
    return pl.BlockSpec((BLK, width), lambda i: (i, 0))


def _full_spec(shape):
    return pl.BlockSpec(shape, lambda i: (0,) * len(shape))


def _selfproj(width):
    return pl.pallas_call(
        _selfproj_body,
        grid=(N // BLK,),
        in_specs=[_row_spec(H), _full_spec((width, H)),
                  _full_spec((1, width))],
        out_specs=_row_spec(width),
        out_shape=jax.ShapeDtypeStruct((N, width), jnp.float32),
    )


_selfproj128 = _selfproj(H)
_selfproj48 = _selfproj(CP)

_dense0 = pl.pallas_call(
    _dense0_body,
    grid=(N // BLK,),
    in_specs=[
        _row_spec(D), _row_spec(D),
        _row_spec(16), _row_spec(16),
        _row_spec(H),
        _full_spec((H, D)), _full_spec((CP, H)),
    ],
    out_specs=[_row_spec(H), _row_spec(CP)],
    out_shape=[jax.ShapeDtypeStruct((N, H), jnp.float32),
               jax.ShapeDtypeStruct((N, CP), jnp.float32)],
)

_final = pl.pallas_call(
    _final_body,
    grid=(N // BLK,),
    in_specs=[
        _row_spec(CP), _row_spec(CP),
        _row_spec(16), _row_spec(16),
        _row_spec(CP),
    ],
    out_specs=_row_spec(CP),
    out_shape=jax.ShapeDtypeStruct((N, CP), jnp.float32),
)


_PAD = E_PAD - E
_SRC_PAD = np.arange(_PAD, dtype=np.int32) * 37 % N
_DST_PAD = (TRASH + np.arange(_PAD, dtype=np.int32) % (N_PAD - N)).astype(
    np.int32)
_ZROWS128 = np.zeros((ROWS_T, D), np.float32)
_ZROWS48 = np.zeros((ROWS_T, CP), np.float32)
_ZDEG = np.zeros((ROWS_T, 16), np.float32)
_ONES_ROWS = np.ones((CH, 16), np.float32)


def kernel(x, edge_index, y, W0_l, b0, W0_r, W1_l, b1, W1_r):
    src1 = jnp.concatenate([edge_index[0], _SRC_PAD]).reshape(
        E_PAD // CH, CH)
    dst1 = jnp.concatenate([edge_index[1], _DST_PAD]).reshape(
        E_PAD // CH, CH)

    src64 = src1.reshape(E_PAD // CH48, CH48)
    dst64 = dst1.reshape(E_PAD // CH48, CH48)

    xr = _selfproj128(x, W0_r, b0.reshape(1, H))
    degA, degB = _sc_deg(dst1, _ONES_ROWS, _ZDEG)
    aggA, aggB = _sc_agg128(x, src64, dst64, _ZROWS128)
    h, hW = _dense0(aggA, aggB, degA, degB, xr, W0_l, W1_l)
    hr = _selfproj48(h, W1_r, b1.reshape(1, C))
    agg1A, agg1B = _sc_agg48(hW, src64, dst64, _ZROWS48)
    return _final(agg1A, agg1B, degA, degB, hr)


# speedup vs baseline: 1.0459x; 1.0227x over previous
import functools

import jax
import jax.numpy as jnp
import numpy as np
from jax import lax
from jax.experimental import pallas as pl
from jax.experimental.pallas import tpu as pltpu
from jax.experimental.pallas import tpu_sc as plsc

N = 10000
E = 320000
D = 128
H = 128
C = 40
CP = 40

NC = 2
NS = 16
NW = NC * NS
CH = 128
NCHUNK = 80
EPT = NCHUNK * CH
E_PAD = NW * EPT
N_PAD = 10240
ROWS_T = N_PAD // NS
TRASH = N

_MESH = plsc.VectorSubcoreMesh(core_axis_name="c", subcore_axis_name="s")
_SC_PARAMS = pltpu.CompilerParams(use_tc_tiling_on_sc=False)


_FULL = N // ROWS_T
_LAST = N - _FULL * ROWS_T


def _drain_stripes(c, s, acc, outA, outB):
    def drain(out, rows, base):
        pltpu.sync_copy(acc.at[pl.ds(base, rows)], out.at[pl.ds(base, rows)])

    for cc, out in ((0, outA), (1, outB)):
        @pl.when(jnp.logical_and(c == cc, s < _FULL))
        def _(out=out):
            drain(out, ROWS_T, s * ROWS_T)
        if _LAST:
            @pl.when(jnp.logical_and(c == cc, s == _FULL))
            def _(out=out):
                drain(out, _LAST, _FULL * ROWS_T)


NHALF = 2
CPH = NCHUNK // NHALF


def _sc_agg(width, ch=CH, nb=2, nstage=NHALF, params=_SC_PARAMS):
    nchunk = EPT // ch
    cps = nchunk // nstage
    ng = cps // nb

    @functools.partial(
        pl.kernel, mesh=_MESH, compiler_params=params,
        out_type=[jax.ShapeDtypeStruct((N, width), jnp.float32),
                  jax.ShapeDtypeStruct((N, width), jnp.float32)],
        scratch_types=[pltpu.VMEM((cps, ch), jnp.int32) for _ in range(4)]
        + [pltpu.VMEM((ch, width), jnp.float32) for _ in range(nb)]
        + [pltpu.VMEM_SHARED((N_PAD, width), jnp.float32)]
        + [pltpu.SemaphoreType.DMA] * (2 * nb + 2))
    def body(feat_hbm, src_hbm, dst_hbm, zrows_hbm, outA, outB, *rest):
        src_b = rest[0:2]
        dst_b = rest[2:4]
        rows = rest[4:4 + nb]
        acc_sh = rest[4 + nb]
        gsem = rest[5 + nb:5 + 2 * nb]
        ssem = rest[5 + 2 * nb:5 + 3 * nb]
        isem = rest[5 + 3 * nb:]
        c = lax.axis_index("c")
        s = lax.axis_index("s")
        wid = c * NS + s

        pltpu.sync_copy(zrows_hbm, acc_sh.at[pl.ds(s * ROWS_T, ROWS_T)])
        plsc.subcore_barrier()

        def prefetch(h, p):
            base = wid * nchunk + h * cps
            pltpu.async_copy(src_hbm.at[pl.ds(base, cps)], src_b[p], isem[p])
            pltpu.async_copy(dst_hbm.at[pl.ds(base, cps)], dst_b[p], isem[p])

        def wait_prefetch(p):
            pltpu.make_async_copy(src_hbm.at[pl.ds(0, cps)], src_b[p],
                                  isem[p]).wait()
            pltpu.make_async_copy(dst_hbm.at[pl.ds(0, cps)], dst_b[p],
                                  isem[p]).wait()

        def gather(j, b, p):
            pltpu.async_copy(feat_hbm.at[src_b[p].at[j]], rows[b], gsem[b])

        def scatter(j, b, p):
            pltpu.async_copy(rows[b], acc_sh.at[dst_b[p].at[j]], ssem[b],
                             add=True)

        def wait_gather(b):
            pltpu.make_async_copy(feat_hbm.at[src_b[0].at[0]], rows[b],
                                  gsem[b]).wait()

        def wait_scatter(b):
            pltpu.make_async_copy(rows[b], acc_sh.at[dst_b[0].at[0]],
                                  ssem[b]).wait()

        def run_groups(p):
            def group(g, carry2):
                for b in range(nb):
                    wait_gather(b)
                    scatter(g * nb + b, b, p)
                for b in range(nb):
                    jn = (g + 1) * nb + b

                    @pl.when(jn < cps)
                    def _(jn=jn, b=b):
                        wait_scatter(b)
                        gather(jn, b, p)
                return carry2

            lax.fori_loop(0, ng, group, 0)

        prefetch(0, 0)
        prefetch(1, 1)
        wait_prefetch(0)
        for b in range(nb):
            gather(b, b, 0)

        def pair(g, carry):
            for p in range(2):
                h = 2 * g + p
                run_groups(p)

                @pl.when(h + 1 < nstage)
                def _(p=p):
                    wait_prefetch(1 - p)
                    for b in range(nb):
                        wait_scatter(b)
                        gather(b, b, 1 - p)

                @pl.when(h + 1 >= nstage)
                def _():
                    for b in range(nb):
                        wait_scatter(b)

                @pl.when(h + 2 < nstage)
                def _(h=h, p=p):
                    prefetch(h + 2, p)
            return carry

        lax.fori_loop(0, nstage // 2, pair, 0)

        plsc.subcore_barrier()
        _drain_stripes(c, s, acc_sh, outA, outB)

    return body


@functools.partial(
    pl.kernel, mesh=_MESH, compiler_params=_SC_PARAMS,
    out_type=[jax.ShapeDtypeStruct((N, 16), jnp.float32),
              jax.ShapeDtypeStruct((N, 16), jnp.float32)],
    scratch_types=[
        pltpu.VMEM((NCHUNK, CH), jnp.int32),
        pltpu.VMEM((CH, 16), jnp.float32),
        pltpu.VMEM_SHARED((N_PAD, 16), jnp.float32),
        pltpu.SemaphoreType.DMA,
    ])
def _sc_deg(dst_hbm, ones_hbm, zdeg_hbm, degA, degB, dst_v, ones_v, deg_sh,
            sem):
    c = lax.axis_index("c")
    s = lax.axis_index("s")
    wid = c * NS + s
    GRP = 8

    pltpu.sync_copy(zdeg_hbm, deg_sh.at[pl.ds(s * ROWS_T, ROWS_T)])
    pltpu.sync_copy(ones_hbm, ones_v)
    pltpu.sync_copy(dst_hbm.at[pl.ds(wid * NCHUNK, NCHUNK)], dst_v)

    plsc.subcore_barrier()

    def fire(g):
        for b in range(GRP):
            pltpu.async_copy(ones_v, deg_sh.at[dst_v.at[g * GRP + b]], sem,
                             add=True)

    def drain():
        for _ in range(GRP):
            pltpu.make_async_copy(ones_v, deg_sh.at[dst_v.at[0]],
                                  sem).wait()

    fire(0)

    def group(g, carry):
        fire(g)
        drain()
        return carry

    lax.fori_loop(1, NCHUNK // GRP, group, 0)
    drain()

    plsc.subcore_barrier()
    _drain_stripes(c, s, deg_sh, degA, degB)


CH48 = 64
_sc_agg128 = _sc_agg(D, ch=CH48, nb=4, nstage=8)
_sc_agg48 = _sc_agg(CP, ch=CH48, nb=4, nstage=8)

BLK = 2000


def _selfproj_body(x, W, b, out):
    dn = (((1,), (1,)), ((), ()))
    out[...] = lax.dot_general(x[...], W[...], dn,
                               preferred_element_type=jnp.float32) + b[...]


def _dense0_body(aggA, aggB, degA, degB, xr, W0l, W1lp, h_out, hW_out):
    deg = jnp.maximum(degA[:, :1] + degB[:, :1], 1.0)
    mean = (aggA[...] + aggB[...]) / deg
    dn = (((1,), (1,)), ((), ()))
    h = lax.dot_general(mean, W0l[...], dn, preferred_element_type=jnp.float32)
    h = jnp.maximum(h + xr[...], 0.0)
    h_out[...] = h
    hW_out[...] = lax.dot_general(h, W1lp[...], dn,
                                  preferred_element_type=jnp.float32)


def _final_body(aggA, aggB, degA, degB, hr, out):
    deg = jnp.maximum(degA[:, :1] + degB[:, :1], 1.0)
    mean = (aggA[...] + aggB[...]) / deg
    z = jnp.maximum(mean + hr[...], 0.0)
    m = jnp.max(z, axis=1, keepdims=True)
    lse = jnp.log(jnp.sum(jnp.exp(z - m), axis=1, keepdims=True)) + m
    out[...] = z - lse


def _row_spec(width):
    return pl.BlockSpec((BLK, width), lambda i: (i, 0))


def _full_spec(shape):
    return pl.BlockSpec(shape, lambda i: (0,) * len(shape))


def _selfproj(width):
    return pl.pallas_call(
        _selfproj_body,
        grid=(N // BLK,),
        in_specs=[_row_spec(H), _full_spec((width, H)),
                  _full_spec((1, width))],
        out_specs=_row_spec(width),
        out_shape=jax.ShapeDtypeStruct((N, width), jnp.float32),
    )


_selfproj128 = _selfproj(H)
_selfproj48 = _selfproj(CP)

_dense0 = pl.pallas_call(
    _dense0_body,
    grid=(N // BLK,),
    in_specs=[
        _row_spec(D), _row_spec(D),
        _row_spec(16), _row_spec(16),
        _row_spec(H),
        _full_spec((H, D)), _full_spec((CP, H)),
    ],
    out_specs=[_row_spec(H), _row_spec(CP)],
    out_shape=[jax.ShapeDtypeStruct((N, H), jnp.float32),
               jax.ShapeDtypeStruct((N, CP), jnp.float32)],
)

_final = pl.pallas_call(
    _final_body,
    grid=(N // BLK,),
    in_specs=[
        _row_spec(CP), _row_spec(CP),
        _row_spec(16), _row_spec(16),
        _row_spec(CP),
    ],
    out_specs=_row_spec(CP),
    out_shape=jax.ShapeDtypeStruct((N, CP), jnp.float32),
)


_PAD = E_PAD - E
_SRC_PAD = np.arange(_PAD, dtype=np.int32) * 37 % N
_DST_PAD = (TRASH + np.arange(_PAD, dtype=np.int32) % (N_PAD - N)).astype(
    np.int32)
_ZROWS128 = np.zeros((ROWS_T, D), np.float32)
_ZROWS48 = np.zeros((ROWS_T, CP), np.float32)
_ZDEG = np.zeros((ROWS_T, 16), np.float32)
_ONES_ROWS = np.ones((CH, 16), np.float32)


def kernel(x, edge_index, y, W0_l, b0, W0_r, W1_l, b1, W1_r):
    src1 = jnp.concatenate([edge_index[0], _SRC_PAD]).reshape(
        E_PAD // CH, CH)
    dst1 = jnp.concatenate([edge_index[1], _DST_PAD]).reshape(
        E_PAD // CH, CH)

    src64 = src1.reshape(E_PAD // CH48, CH48)
    dst64 = dst1.reshape(E_PAD // CH48, CH48)

    xr = _selfproj128(x, W0_r, b0.reshape(1, H))
    degA, degB = _sc_deg(dst1, _ONES_ROWS, _ZDEG)
    aggA, aggB = _sc_agg128(x, src64, dst64, _ZROWS128)
    h, hW = _dense0(aggA, aggB, degA, degB, xr, W0_l, W1_l)
    hr = _selfproj48(h, W1_r, b1.reshape(1, C))
    agg1A, agg1B = _sc_agg48(hW, src64, dst64, _ZROWS48)
    return _final(agg1A, agg1B, degA, degB, hr)


# speedup vs baseline: 1.0484x; 1.0024x over previous
import functools

import jax
import jax.numpy as jnp
import numpy as np
from jax import lax
from jax.experimental import pallas as pl
from jax.experimental.pallas import tpu as pltpu
from jax.experimental.pallas import tpu_sc as plsc

N = 10000
E = 320000
D = 128
H = 128
C = 40
CP = 40

NC = 2
NS = 16
NW = NC * NS
CH = 128
NCHUNK = 80
EPT = NCHUNK * CH
E_PAD = NW * EPT
N_PAD = 10240
ROWS_T = N_PAD // NS
TRASH = N

_MESH = plsc.VectorSubcoreMesh(core_axis_name="c", subcore_axis_name="s")
_SC_PARAMS = pltpu.CompilerParams(use_tc_tiling_on_sc=False)


_FULL = N // ROWS_T
_LAST = N - _FULL * ROWS_T


def _drain_stripes(c, s, acc, outA, outB):
    def drain(out, rows, base):
        pltpu.sync_copy(acc.at[pl.ds(base, rows)], out.at[pl.ds(base, rows)])

    for cc, out in ((0, outA), (1, outB)):
        @pl.when(jnp.logical_and(c == cc, s < _FULL))
        def _(out=out):
            drain(out, ROWS_T, s * ROWS_T)
        if _LAST:
            @pl.when(jnp.logical_and(c == cc, s == _FULL))
            def _(out=out):
                drain(out, _LAST, _FULL * ROWS_T)


NHALF = 2
CPH = NCHUNK // NHALF


def _sc_agg(width, ch=CH, nb=2, nstage=NHALF, params=_SC_PARAMS):
    nchunk = EPT // ch
    cps = nchunk // nstage
    ng = cps // nb

    @functools.partial(
        pl.kernel, mesh=_MESH, compiler_params=params,
        out_type=[jax.ShapeDtypeStruct((N, width), jnp.float32),
                  jax.ShapeDtypeStruct((N, width), jnp.float32)],
        scratch_types=[pltpu.VMEM((cps, ch), jnp.int32) for _ in range(4)]
        + [pltpu.VMEM((ch, width), jnp.float32) for _ in range(nb)]
        + [pltpu.VMEM_SHARED((N_PAD, width), jnp.float32)]
        + [pltpu.SemaphoreType.DMA] * (2 * nb + 2))
    def body(feat_hbm, src_hbm, dst_hbm, zrows_hbm, outA, outB, *rest):
        src_b = rest[0:2]
        dst_b = rest[2:4]
        rows = rest[4:4 + nb]
        acc_sh = rest[4 + nb]
        gsem = rest[5 + nb:5 + 2 * nb]
        ssem = rest[5 + 2 * nb:5 + 3 * nb]
        isem = rest[5 + 3 * nb:]
        c = lax.axis_index("c")
        s = lax.axis_index("s")
        wid = c * NS + s

        pltpu.sync_copy(zrows_hbm, acc_sh.at[pl.ds(s * ROWS_T, ROWS_T)])
        plsc.subcore_barrier()

        def prefetch(h, p):
            base = wid * nchunk + h * cps
            pltpu.async_copy(src_hbm.at[pl.ds(base, cps)], src_b[p], isem[p])
            pltpu.async_copy(dst_hbm.at[pl.ds(base, cps)], dst_b[p], isem[p])

        def wait_prefetch(p):
            pltpu.make_async_copy(src_hbm.at[pl.ds(0, cps)], src_b[p],
                                  isem[p]).wait()
            pltpu.make_async_copy(dst_hbm.at[pl.ds(0, cps)], dst_b[p],
                                  isem[p]).wait()

        def gather(j, b, p):
            pltpu.async_copy(feat_hbm.at[src_b[p].at[j]], rows[b], gsem[b])

        def scatter(j, b, p):
            pltpu.async_copy(rows[b], acc_sh.at[dst_b[p].at[j]], ssem[b],
                             add=True)

        def wait_gather(b):
            pltpu.make_async_copy(feat_hbm.at[src_b[0].at[0]], rows[b],
                                  gsem[b]).wait()

        def wait_scatter(b):
            pltpu.make_async_copy(rows[b], acc_sh.at[dst_b[0].at[0]],
                                  ssem[b]).wait()

        def run_groups(p):
            def group(g, carry2):
                for b in range(nb):
                    wait_gather(b)
                    scatter(g * nb + b, b, p)
                for b in range(nb):
                    jn = (g + 1) * nb + b

                    @pl.when(jn < cps)
                    def _(jn=jn, b=b):
                        wait_scatter(b)
                        gather(jn, b, p)
                return carry2

            lax.fori_loop(0, ng, group, 0)

        prefetch(0, 0)
        prefetch(1, 1)
        wait_prefetch(0)
        for b in range(nb):
            gather(b, b, 0)

        def pair(g, carry):
            for p in range(2):
                h = 2 * g + p
                run_groups(p)

                @pl.when(h + 1 < nstage)
                def _(p=p):
                    wait_prefetch(1 - p)
                    for b in range(nb):
                        wait_scatter(b)
                        gather(b, b, 1 - p)

                @pl.when(h + 1 >= nstage)
                def _():
                    for b in range(nb):
                        wait_scatter(b)

                @pl.when(h + 2 < nstage)
                def _(h=h, p=p):
                    prefetch(h + 2, p)
            return carry

        lax.fori_loop(0, nstage // 2, pair, 0)

        plsc.subcore_barrier()
        _drain_stripes(c, s, acc_sh, outA, outB)

    return body


@functools.partial(
    pl.kernel, mesh=_MESH, compiler_params=_SC_PARAMS,
    out_type=[jax.ShapeDtypeStruct((N, 16), jnp.float32),
              jax.ShapeDtypeStruct((N, 16), jnp.float32)],
    scratch_types=[
        pltpu.VMEM((NCHUNK, CH), jnp.int32),
        pltpu.VMEM((CH, 16), jnp.float32),
        pltpu.VMEM_SHARED((N_PAD, 16), jnp.float32),
        pltpu.SemaphoreType.DMA,
    ])
def _sc_deg(dst_hbm, ones_hbm, zdeg_hbm, degA, degB, dst_v, ones_v, deg_sh,
            sem):
    c = lax.axis_index("c")
    s = lax.axis_index("s")
    wid = c * NS + s
    GRP = 16

    pltpu.sync_copy(zdeg_hbm, deg_sh.at[pl.ds(s * ROWS_T, ROWS_T)])
    pltpu.sync_copy(ones_hbm, ones_v)
    pltpu.sync_copy(dst_hbm.at[pl.ds(wid * NCHUNK, NCHUNK)], dst_v)

    plsc.subcore_barrier()

    def fire(g):
        for b in range(GRP):
            pltpu.async_copy(ones_v, deg_sh.at[dst_v.at[g * GRP + b]], sem,
                             add=True)

    def drain():
        for _ in range(GRP):
            pltpu.make_async_copy(ones_v, deg_sh.at[dst_v.at[0]],
                                  sem).wait()

    fire(0)

    def group(g, carry):
        fire(g)
        drain()
        return carry

    lax.fori_loop(1, NCHUNK // GRP, group, 0)
    drain()

    plsc.subcore_barrier()
    _drain_stripes(c, s, deg_sh, degA, degB)


CH48 = 64
_sc_agg128 = _sc_agg(D, ch=CH48, nb=4, nstage=8)
_sc_agg48 = _sc_agg(CP, ch=CH48, nb=4, nstage=8)

BLK = 2000


def _selfproj_body(x, W, b, out):
    dn = (((1,), (1,)), ((), ()))
    out[...] = lax.dot_general(x[...], W[...], dn,
                               preferred_element_type=jnp.float32) + b[...]


def _dense0_body(aggA, aggB, degA, degB, xr, W0l, W1lp, h_out, hW_out):
    deg = jnp.maximum(degA[:, :1] + degB[:, :1], 1.0)
    mean = (aggA[...] + aggB[...]) / deg
    dn = (((1,), (1,)), ((), ()))
    h = lax.dot_general(mean, W0l[...], dn, preferred_element_type=jnp.float32)
    h = jnp.maximum(h + xr[...], 0.0)
    h_out[...] = h
    hW_out[...] = lax.dot_general(h, W1lp[...], dn,
                                  preferred_element_type=jnp.float32)


def _final_body(aggA, aggB, degA, degB, hr, out):
    deg = jnp.maximum(degA[:, :1] + degB[:, :1], 1.0)
    mean = (aggA[...] + aggB[...]) / deg
    z = jnp.maximum(mean + hr[...], 0.0)
    m = jnp.max(z, axis=1, keepdims=True)
    lse = jnp.log(jnp.sum(jnp.exp(z - m), axis=1, keepdims=True)) + m
    out[...] = z - lse


def _row_spec(width):
    return pl.BlockSpec((BLK, width), lambda i: (i, 0))


def _full_spec(shape):
    return pl.BlockSpec(shape, lambda i: (0,) * len(shape))


def _selfproj(width):
    return pl.pallas_call(
        _selfproj_body,
        grid=(N // BLK,),
        in_specs=[_row_spec(H), _full_spec((width, H)),
                  _full_spec((1, width))],
        out_specs=_row_spec(width),
        out_shape=jax.ShapeDtypeStruct((N, width), jnp.float32),
    )


_selfproj128 = _selfproj(H)
_selfproj48 = _selfproj(CP)

_dense0 = pl.pallas_call(
    _dense0_body,
    grid=(N // BLK,),
    in_specs=[
        _row_spec(D), _row_spec(D),
        _row_spec(16), _row_spec(16),
        _row_spec(H),
        _full_spec((H, D)), _full_spec((CP, H)),
    ],
    out_specs=[_row_spec(H), _row_spec(CP)],
    out_shape=[jax.ShapeDtypeStruct((N, H), jnp.float32),
               jax.ShapeDtypeStruct((N, CP), jnp.float32)],
)

_final = pl.pallas_call(
    _final_body,
    grid=(N // BLK,),
    in_specs=[
        _row_spec(CP), _row_spec(CP),
        _row_spec(16), _row_spec(16),
        _row_spec(CP),
    ],
    out_specs=_row_spec(CP),
    out_shape=jax.ShapeDtypeStruct((N, CP), jnp.float32),
)


_PAD = E_PAD - E
_SRC_PAD = np.arange(_PAD, dtype=np.int32) * 37 % N
_DST_PAD = (TRASH + np.arange(_PAD, dtype=np.int32) % (N_PAD - N)).astype(
    np.int32)
_ZROWS128 = np.zeros((ROWS_T, D), np.float32)
_ZROWS48 = np.zeros((ROWS_T, CP), np.float32)
_ZDEG = np.zeros((ROWS_T, 16), np.float32)
_ONES_ROWS = np.ones((CH, 16), np.float32)


def kernel(x, edge_index, y, W0_l, b0, W0_r, W1_l, b1, W1_r):
    src1 = jnp.concatenate([edge_index[0], _SRC_PAD]).reshape(
        E_PAD // CH, CH)
    dst1 = jnp.concatenate([edge_index[1], _DST_PAD]).reshape(
        E_PAD // CH, CH)

    src64 = src1.reshape(E_PAD // CH48, CH48)
    dst64 = dst1.reshape(E_PAD // CH48, CH48)

    xr = _selfproj128(x, W0_r, b0.reshape(1, H))
    degA, degB = _sc_deg(dst1, _ONES_ROWS, _ZDEG)
    aggA, aggB = _sc_agg128(x, src64, dst64, _ZROWS128)
    h, hW = _dense0(aggA, aggB, degA, degB, xr, W0_l, W1_l)
    hr = _selfproj48(h, W1_r, b1.reshape(1, C))
    agg1A, agg1B = _sc_agg48(hW, src64, dst64, _ZROWS48)
    return _final(agg1A, agg1B, degA, degB, hr)
